# baseline (speedup 1.0000x reference)
import jax
import jax.numpy as jnp
from jax import lax
from jax.experimental import pallas as pl
from jax.experimental.pallas import tpu as pltpu
from jax.experimental.pallas import tpu_sc as plsc

N = 10000
E = 320000
F = 128
FH = F // 2

NC = 2
NS = 16
NW = NC * NS
CH = 128
NBUF = 4
CPT = 160
NCHUNK = NS * CPT
EP = NCHUNK * CH
DCPT = NCHUNK // NW
NP = 12800
NPA = 10112
RPT = NPA // NS

_mesh = plsc.VectorSubcoreMesh(core_axis_name="c", subcore_axis_name="s",
                               num_cores=NC, num_subcores=NS)


def _deg_body(dst_hbm, ones_hbm, zeros_hbm, deg_hbm,
              idx_v, ones_v, acc_sh):
    c = lax.axis_index("c")
    s = lax.axis_index("s")
    w = c * NS + s
    pltpu.sync_copy(zeros_hbm, acc_sh.at[pl.ds(s * RPT, RPT)])
    pltpu.sync_copy(ones_hbm, ones_v)
    pltpu.sync_copy(dst_hbm.at[pl.ds(w * DCPT, DCPT)], idx_v)
    plsc.subcore_barrier()

    def step(t, carry):
        pltpu.sync_copy(ones_v, acc_sh.at[idx_v.at[t]], add=True)
        return carry

    lax.fori_loop(0, DCPT, step, 0)
    plsc.subcore_barrier()
    pltpu.sync_copy(acc_sh.at[pl.ds(s * RPT, RPT)],
                    deg_hbm.at[pl.ds(c * NP + s * RPT, RPT)])


def _agg_body(src_hbm, dst_hbm, x_hbm, zeros_hbm, part_hbm,
              src_v, dst_v, r0, r1, r2, r3, acc_sh,
              sem0, sem1, sem2, sem3, ssem0, ssem1, ssem2, ssem3):
    c = lax.axis_index("c")
    s = lax.axis_index("s")
    rows = [r0, r1, r2, r3]
    sems = [sem0, sem1, sem2, sem3]
    ssems = [ssem0, ssem1, ssem2, ssem3]
    pltpu.sync_copy(zeros_hbm, acc_sh.at[pl.ds(s * RPT, RPT)])
    pltpu.sync_copy(src_hbm.at[pl.ds(c * NCHUNK + s * CPT, CPT)], src_v)
    pltpu.sync_copy(dst_hbm.at[pl.ds(s * CPT, CPT)], dst_v)
    plsc.subcore_barrier()

    def gather(t, b):
        return pltpu.make_async_copy(x_hbm.at[src_v.at[t]], rows[b], sems[b])

    def scat_start(t, b):
        pltpu.async_copy(rows[b], acc_sh.at[dst_v.at[t]], ssems[b], add=True)

    def scat_wait(t, b):
        pltpu.make_async_copy(rows[b], acc_sh.at[dst_v.at[t]], ssems[b]).wait()

    for b in range(NBUF):
        gather(b, b).start()

    def step(g, carry):
        for b in range(NBUF):
            t = g * NBUF + b
            gather(t, b).wait()
            scat_start(t, b)
        for b in range(NBUF):
            t = g * NBUF + b
            scat_wait(t, b)
            gather(t + NBUF, b).start()
        return carry

    lax.fori_loop(0, CPT // NBUF - 1, step, 0)
    for b in range(NBUF):
        t = CPT - NBUF + b
        gather(t, b).wait()
        scat_start(t, b)
    for b in range(NBUF):
        scat_wait(CPT - NBUF + b, b)

    plsc.subcore_barrier()
    pltpu.sync_copy(acc_sh.at[pl.ds(s * RPT, RPT)],
                    part_hbm.at[pl.ds(c * NP + s * RPT, RPT)])


_sc_params = pltpu.CompilerParams(use_tc_tiling_on_sc=False)

_deg_call = pl.kernel(
    _deg_body,
    out_type=jax.ShapeDtypeStruct((2 * NP, 16), jnp.float32),
    mesh=_mesh,
    compiler_params=_sc_params,
    scratch_types=[
        pltpu.VMEM((DCPT, CH), jnp.int32),
        pltpu.VMEM((CH, 16), jnp.float32),
        pltpu.VMEM_SHARED((NPA, 16), jnp.float32),
    ],
)

_agg_call = pl.kernel(
    _agg_body,
    out_type=jax.ShapeDtypeStruct((2 * NP, FH), jnp.float32),
    mesh=_mesh,
    compiler_params=_sc_params,
    scratch_types=[
        pltpu.VMEM((CPT, CH), jnp.int32),
        pltpu.VMEM((CPT, CH), jnp.int32),
        pltpu.VMEM((CH, FH), jnp.float32),
        pltpu.VMEM((CH, FH), jnp.float32),
        pltpu.VMEM((CH, FH), jnp.float32),
        pltpu.VMEM((CH, FH), jnp.float32),
        pltpu.VMEM_SHARED((NPA, FH), jnp.float32),
        pltpu.SemaphoreType.DMA,
        pltpu.SemaphoreType.DMA,
        pltpu.SemaphoreType.DMA,
        pltpu.SemaphoreType.DMA,
        pltpu.SemaphoreType.DMA,
        pltpu.SemaphoreType.DMA,
        pltpu.SemaphoreType.DMA,
        pltpu.SemaphoreType.DMA,
    ],
)


_R = 400
_G = N // _R
_O = NP // _R


def _prep_body(d0_ref, d1_ref, x_ref, dis_ref, s2_ref):
    dsum = d0_ref[...] + d1_ref[...]
    deg = jnp.maximum(dsum[:, 0:1], 1.0)
    dis = lax.rsqrt(deg)
    dis_ref[...] = dis
    s0 = x_ref[...] * dis
    s2_ref[0] = s0[:, :FH]
    s2_ref[1] = s0[:, FH:]


_prep_call = pl.pallas_call(
    _prep_body,
    grid=(_G,),
    in_specs=[
        pl.BlockSpec((_R, 16), lambda i: (i, 0)),
        pl.BlockSpec((_R, 16), lambda i: (i + _O, 0)),
        pl.BlockSpec((_R, F), lambda i: (i, 0)),
    ],
    out_specs=[
        pl.BlockSpec((_R, 1), lambda i: (i, 0)),
        pl.BlockSpec((2, _R, FH), lambda i: (0, i, 0)),
    ],
    out_shape=[
        jax.ShapeDtypeStruct((N, 1), jnp.float32),
        jax.ShapeDtypeStruct((2, N, FH), jnp.float32),
    ],
)


def _layer_body(plo_ref, phi_ref, dis_ref, w_ref, y_ref, snext_ref):
    agg = jnp.concatenate([plo_ref[...], phi_ref[...]], axis=1)
    a = agg * dis_ref[...]
    y = jnp.dot(a, w_ref[...], preferred_element_type=jnp.float32)
    y_ref[...] = y
    s = jnp.maximum(y, 0.0) * dis_ref[...]
    snext_ref[0] = s[:, :FH]
    snext_ref[1] = s[:, FH:]


_layer_call = pl.pallas_call(
    _layer_body,
    grid=(_G,),
    in_specs=[
        pl.BlockSpec((_R, FH), lambda i: (i, 0)),
        pl.BlockSpec((_R, FH), lambda i: (i + _O, 0)),
        pl.BlockSpec((_R, 1), lambda i: (i, 0)),
        pl.BlockSpec((F, F), lambda i: (0, 0)),
    ],
    out_specs=[
        pl.BlockSpec((_R, F), lambda i: (i, 0)),
        pl.BlockSpec((2, _R, FH), lambda i: (0, i, 0)),
    ],
    out_shape=[
        jax.ShapeDtypeStruct((N, F), jnp.float32),
        jax.ShapeDtypeStruct((2, N, FH), jnp.float32),
    ],
)


def kernel(graph, features, W1, W2, W3):
    srcp = jnp.pad(graph[0], (0, EP - E))
    dstp = jnp.pad(graph[1], (0, EP - E), constant_values=N)
    src2 = jnp.concatenate([srcp, srcp + N]).reshape(2 * NCHUNK, CH)
    dst2 = dstp.reshape(NCHUNK, CH)
    ones_d = jnp.ones((CH, 16), jnp.float32)
    zeros_d = jnp.zeros((RPT, 16), jnp.float32)
    zeros_f = jnp.zeros((RPT, FH), jnp.float32)
    deg = _deg_call(dst2, ones_d, zeros_d)
    dis, s2 = _prep_call(deg, deg, features)

    def step(carry, W):
        s2, _ = carry
        part = _agg_call(src2, dst2, s2.reshape(2 * N, FH), zeros_f)
        y, s2_next = _layer_call(part, part, dis, W)
        return (s2_next, y), None

    y0 = jnp.zeros((N, F), jnp.float32)
    (_, y), _ = lax.scan(step, (s2, y0), jnp.stack([W1, W2, W3]))
    return y


# speedup vs baseline: 1.0494x; 1.0494x over previous
import jax
import jax.numpy as jnp
from jax import lax
from jax.experimental import pallas as pl
from jax.experimental.pallas import tpu as pltpu
from jax.experimental.pallas import tpu_sc as plsc

N = 10000
E = 320000
F = 128
FH = F // 2

NC = 2
NS = 16
NW = NC * NS
CH = 128
NBUF = 5
CPT = 160
NCHUNK = NS * CPT
EP = NCHUNK * CH
DCPT = NCHUNK // NW
NP = 12800
NPA = 10112
RPT = NPA // NS

_mesh = plsc.VectorSubcoreMesh(core_axis_name="c", subcore_axis_name="s",
                               num_cores=NC, num_subcores=NS)


def _deg_body(dst_hbm, ones_hbm, zeros_hbm, deg_hbm,
              idx_v, ones_v, acc_sh):
    c = lax.axis_index("c")
    s = lax.axis_index("s")
    w = c * NS + s
    pltpu.sync_copy(zeros_hbm, acc_sh.at[pl.ds(s * RPT, RPT)])
    pltpu.sync_copy(ones_hbm, ones_v)
    pltpu.sync_copy(dst_hbm.at[pl.ds(w * DCPT, DCPT)], idx_v)
    plsc.subcore_barrier()

    def step(t, carry):
        pltpu.sync_copy(ones_v, acc_sh.at[idx_v.at[t]], add=True)
        return carry

    lax.fori_loop(0, DCPT, step, 0)
    plsc.subcore_barrier()
    pltpu.sync_copy(acc_sh.at[pl.ds(s * RPT, RPT)],
                    deg_hbm.at[pl.ds(c * NP + s * RPT, RPT)])


def _agg_body(src_hbm, dst_hbm, x_hbm, zeros_hbm, part_hbm,
              src_v, dst_v, r0, r1, r2, r3, r4, acc_sh,
              sem0, sem1, sem2, sem3, sem4,
              ssem0, ssem1, ssem2, ssem3, ssem4):
    c = lax.axis_index("c")
    s = lax.axis_index("s")
    rows = [r0, r1, r2, r3, r4]
    sems = [sem0, sem1, sem2, sem3, sem4]
    ssems = [ssem0, ssem1, ssem2, ssem3, ssem4]
    pltpu.sync_copy(zeros_hbm, acc_sh.at[pl.ds(s * RPT, RPT)])
    pltpu.sync_copy(src_hbm.at[pl.ds(c * NCHUNK + s * CPT, CPT)], src_v)
    pltpu.sync_copy(dst_hbm.at[pl.ds(s * CPT, CPT)], dst_v)
    plsc.subcore_barrier()

    def gather(t, b):
        return pltpu.make_async_copy(x_hbm.at[src_v.at[t]], rows[b], sems[b])

    def scat_start(t, b):
        pltpu.async_copy(rows[b], acc_sh.at[dst_v.at[t]], ssems[b], add=True)

    def scat_wait(t, b):
        pltpu.make_async_copy(rows[b], acc_sh.at[dst_v.at[t]], ssems[b]).wait()

    for t in range(3):
        gather(t, t).start()
    for t in range(NBUF):
        gather(t, t).wait()
        scat_start(t, t)
        if t >= 2:
            scat_wait(t - 2, t - 2)
        gather(t + 3, (t + 3) % NBUF).start()

    def step(g, carry):
        for b in range(NBUF):
            t = g * NBUF + b
            gather(t, b).wait()
            scat_start(t, b)
            scat_wait(t - 2, (b - 2) % NBUF)
            gather(t + 3, (b + 3) % NBUF).start()
        return carry

    lax.fori_loop(1, CPT // NBUF - 1, step, 0)
    for b in range(NBUF):
        t = CPT - NBUF + b
        gather(t, b).wait()
        scat_start(t, b)
        scat_wait(t - 2, (b - 2) % NBUF)
        if t + 3 <= CPT - 1:
            gather(t + 3, (b + 3) % NBUF).start()
    for t in range(CPT - 2, CPT):
        scat_wait(t, t % NBUF)

    plsc.subcore_barrier()
    pltpu.sync_copy(acc_sh.at[pl.ds(s * RPT, RPT)],
                    part_hbm.at[pl.ds(c * NP + s * RPT, RPT)])


_sc_params = pltpu.CompilerParams(use_tc_tiling_on_sc=False)

_deg_call = pl.kernel(
    _deg_body,
    out_type=jax.ShapeDtypeStruct((2 * NP, 16), jnp.float32),
    mesh=_mesh,
    compiler_params=_sc_params,
    scratch_types=[
        pltpu.VMEM((DCPT, CH), jnp.int32),
        pltpu.VMEM((CH, 16), jnp.float32),
        pltpu.VMEM_SHARED((NPA, 16), jnp.float32),
    ],
)

_agg_call = pl.kernel(
    _agg_body,
    out_type=jax.ShapeDtypeStruct((2 * NP, FH), jnp.float32),
    mesh=_mesh,
    compiler_params=_sc_params,
    scratch_types=[
        pltpu.VMEM((CPT, CH), jnp.int32),
        pltpu.VMEM((CPT, CH), jnp.int32),
        pltpu.VMEM((CH, FH), jnp.float32),
        pltpu.VMEM((CH, FH), jnp.float32),
        pltpu.VMEM((CH, FH), jnp.float32),
        pltpu.VMEM((CH, FH), jnp.float32),
        pltpu.VMEM((CH, FH), jnp.float32),
        pltpu.VMEM_SHARED((NPA, FH), jnp.float32),
        pltpu.SemaphoreType.DMA,
        pltpu.SemaphoreType.DMA,
        pltpu.SemaphoreType.DMA,
        pltpu.SemaphoreType.DMA,
        pltpu.SemaphoreType.DMA,
        pltpu.SemaphoreType.DMA,
        pltpu.SemaphoreType.DMA,
        pltpu.SemaphoreType.DMA,
        pltpu.SemaphoreType.DMA,
        pltpu.SemaphoreType.DMA,
    ],
)


_R = 400
_G = N // _R
_O = NP // _R


def _prep_body(d0_ref, d1_ref, x_ref, dis_ref, s2_ref):
    dsum = d0_ref[...] + d1_ref[...]
    deg = jnp.maximum(dsum[:, 0:1], 1.0)
    dis = lax.rsqrt(deg)
    dis_ref[...] = dis
    s0 = x_ref[...] * dis
    s2_ref[0] = s0[:, :FH]
    s2_ref[1] = s0[:, FH:]


_prep_call = pl.pallas_call(
    _prep_body,
    grid=(_G,),
    in_specs=[
        pl.BlockSpec((_R, 16), lambda i: (i, 0)),
        pl.BlockSpec((_R, 16), lambda i: (i + _O, 0)),
        pl.BlockSpec((_R, F), lambda i: (i, 0)),
    ],
    out_specs=[
        pl.BlockSpec((_R, 1), lambda i: (i, 0)),
        pl.BlockSpec((2, _R, FH), lambda i: (0, i, 0)),
    ],
    out_shape=[
        jax.ShapeDtypeStruct((N, 1), jnp.float32),
        jax.ShapeDtypeStruct((2, N, FH), jnp.float32),
    ],
)


def _layer_body(plo_ref, phi_ref, dis_ref, w_ref, y_ref, snext_ref):
    agg = jnp.concatenate([plo_ref[...], phi_ref[...]], axis=1)
    a = agg * dis_ref[...]
    y = jnp.dot(a, w_ref[...], preferred_element_type=jnp.float32)
    y_ref[...] = y
    s = jnp.maximum(y, 0.0) * dis_ref[...]
    snext_ref[0] = s[:, :FH]
    snext_ref[1] = s[:, FH:]


_layer_call = pl.pallas_call(
    _layer_body,
    grid=(_G,),
    in_specs=[
        pl.BlockSpec((_R, FH), lambda i: (i, 0)),
        pl.BlockSpec((_R, FH), lambda i: (i + _O, 0)),
        pl.BlockSpec((_R, 1), lambda i: (i, 0)),
        pl.BlockSpec((F, F), lambda i: (0, 0)),
    ],
    out_specs=[
        pl.BlockSpec((_R, F), lambda i: (i, 0)),
        pl.BlockSpec((2, _R, FH), lambda i: (0, i, 0)),
    ],
    out_shape=[
        jax.ShapeDtypeStruct((N, F), jnp.float32),
        jax.ShapeDtypeStruct((2, N, FH), jnp.float32),
    ],
)


def kernel(graph, features, W1, W2, W3):
    srcp = jnp.pad(graph[0], (0, EP - E))
    padv = N + jnp.arange(EP - E, dtype=jnp.int32) % (NPA - N)
    dstp = jnp.concatenate([graph[1], padv])
    src2 = jnp.concatenate([srcp, srcp + N]).reshape(2 * NCHUNK, CH)
    dst2 = dstp.reshape(NCHUNK, CH)
    ones_d = jnp.ones((CH, 16), jnp.float32)
    zeros_d = jnp.zeros((RPT, 16), jnp.float32)
    zeros_f = jnp.zeros((RPT, FH), jnp.float32)
    deg = _deg_call(dst2, ones_d, zeros_d)
    dis, s2 = _prep_call(deg, deg, features)

    def step(carry, W):
        s2, _ = carry
        part = _agg_call(src2, dst2, s2.reshape(2 * N, FH), zeros_f)
        y, s2_next = _layer_call(part, part, dis, W)
        return (s2_next, y), None

    y0 = jnp.zeros((N, F), jnp.float32)
    (_, y), _ = lax.scan(step, (s2, y0), jnp.stack([W1, W2, W3]))
    return y


# speedup vs baseline: 1.8494x; 1.7623x over previous
import jax
import jax.numpy as jnp
from jax import lax
from jax.experimental import pallas as pl
from jax.experimental.pallas import tpu as pltpu
from jax.experimental.pallas import tpu_sc as plsc

N = 10000
E = 320000
F = 128
FH = F // 2

NC = 2
NS = 16
NW = NC * NS
CH = 128
NBUF = 3
CPT = 162
NCHUNK = NS * CPT
EP = NCHUNK * CH
DCPT = NCHUNK // NW
NP = 12800
NPA = 10112
RPT = NPA // NS

_mesh = plsc.VectorSubcoreMesh(core_axis_name="c", subcore_axis_name="s",
                               num_cores=NC, num_subcores=NS)


def _deg_body(dst_hbm, ones_hbm, zeros_hbm, deg_hbm,
              idx_v, ones_v, acc_sh):
    c = lax.axis_index("c")
    s = lax.axis_index("s")
    w = c * NS + s
    pltpu.sync_copy(zeros_hbm, acc_sh.at[pl.ds(s * RPT, RPT)])
    pltpu.sync_copy(ones_hbm, ones_v)
    pltpu.sync_copy(dst_hbm.at[pl.ds(w * DCPT, DCPT)], idx_v)
    plsc.subcore_barrier()

    def step(t, carry):
        pltpu.sync_copy(ones_v, acc_sh.at[idx_v.at[t]], add=True)
        return carry

    lax.fori_loop(0, DCPT, step, 0)
    plsc.subcore_barrier()
    pltpu.sync_copy(acc_sh.at[pl.ds(s * RPT, RPT)],
                    deg_hbm.at[pl.ds(c * NP + s * RPT, RPT)])


def _agg_body(src_hbm, dst_hbm, x_hbm, zeros_hbm, part_hbm,
              src_v, d0, d1, d2, r0, r1, r2, x_sh, acc_sh,
              gs0, gs1, gs2, ss0, ss1, ss2, ds0, ds1, ds2):
    c = lax.axis_index("c")
    s = lax.axis_index("s")
    rows = [r0, r1, r2]
    dsts = [d0, d1, d2]
    gsems = [gs0, gs1, gs2]
    ssems = [ss0, ss1, ss2]
    dsems = [ds0, ds1, ds2]
    pltpu.sync_copy(zeros_hbm, acc_sh.at[pl.ds(s * RPT, RPT)])
    pltpu.sync_copy(x_hbm.at[pl.ds(c * NPA + s * RPT, RPT)],
                    x_sh.at[pl.ds(s * RPT, RPT)])
    pltpu.sync_copy(src_hbm.at[pl.ds(s * CPT, CPT)], src_v)
    plsc.subcore_barrier()
    tbase = s * CPT

    def gather(t, b):
        return pltpu.make_async_copy(x_sh.at[src_v.at[t]], rows[b], gsems[b])

    def dload(t, b):
        return pltpu.make_async_copy(dst_hbm.at[tbase + t], dsts[b], dsems[b])

    def scat_start(t, b):
        pltpu.async_copy(rows[b], acc_sh.at[dsts[b]], ssems[b], add=True)

    def scat_wait(t, b):
        pltpu.make_async_copy(rows[b], acc_sh.at[dsts[b]], ssems[b]).wait()

    for t in range(2):
        dload(t, t).start()
        gather(t, t).start()
    for t in range(NBUF):
        gather(t, t).wait()
        dload(t, t).wait()
        scat_start(t, t)
        if t >= 1:
            scat_wait(t - 1, t - 1)
        dload(t + 2, (t + 2) % NBUF).start()
        gather(t + 2, (t + 2) % NBUF).start()

    def step(g, carry):
        for b in range(NBUF):
            t = g * NBUF + b
            gather(t, b).wait()
            dload(t, b).wait()
            scat_start(t, b)
            scat_wait(t - 1, (b - 1) % NBUF)
            dload(t + 2, (b + 2) % NBUF).start()
            gather(t + 2, (b + 2) % NBUF).start()
        return carry

    lax.fori_loop(1, CPT // NBUF - 1, step, 0)
    for b in range(NBUF):
        t = CPT - NBUF + b
        gather(t, b).wait()
        dload(t, b).wait()
        scat_start(t, b)
        scat_wait(t - 1, (b - 1) % NBUF)
        if t + 2 <= CPT - 1:
            dload(t + 2, (b + 2) % NBUF).start()
            gather(t + 2, (b + 2) % NBUF).start()
    scat_wait(CPT - 1, (CPT - 1) % NBUF)

    plsc.subcore_barrier()
    pltpu.sync_copy(acc_sh.at[pl.ds(s * RPT, RPT)],
                    part_hbm.at[pl.ds(c * NP + s * RPT, RPT)])


_sc_params = pltpu.CompilerParams(use_tc_tiling_on_sc=False)

_deg_call = pl.kernel(
    _deg_body,
    out_type=jax.ShapeDtypeStruct((2 * NP, 16), jnp.float32),
    mesh=_mesh,
    compiler_params=_sc_params,
    scratch_types=[
        pltpu.VMEM((DCPT, CH), jnp.int32),
        pltpu.VMEM((CH, 16), jnp.float32),
        pltpu.VMEM_SHARED((NPA, 16), jnp.float32),
    ],
)

_agg_call = pl.kernel(
    _agg_body,
    out_type=jax.ShapeDtypeStruct((2 * NP, FH), jnp.float32),
    mesh=_mesh,
    compiler_params=_sc_params,
    scratch_types=[
        pltpu.VMEM((CPT, CH), jnp.int32),
        pltpu.VMEM((CH,), jnp.int32),
        pltpu.VMEM((CH,), jnp.int32),
        pltpu.VMEM((CH,), jnp.int32),
        pltpu.VMEM((CH, FH), jnp.float32),
        pltpu.VMEM((CH, FH), jnp.float32),
        pltpu.VMEM((CH, FH), jnp.float32),
        pltpu.VMEM_SHARED((NPA, FH), jnp.float32),
        pltpu.VMEM_SHARED((NPA, FH), jnp.float32),
        pltpu.SemaphoreType.DMA,
        pltpu.SemaphoreType.DMA,
        pltpu.SemaphoreType.DMA,
        pltpu.SemaphoreType.DMA,
        pltpu.SemaphoreType.DMA,
        pltpu.SemaphoreType.DMA,
        pltpu.SemaphoreType.DMA,
        pltpu.SemaphoreType.DMA,
        pltpu.SemaphoreType.DMA,
    ],
)


_R = 400
_G = N // _R
_O = NP // _R


def _prep_body(d0_ref, d1_ref, x_ref, dis_ref, s2_ref):
    dsum = d0_ref[...] + d1_ref[...]
    deg = jnp.maximum(dsum[:, 0:1], 1.0)
    dis = lax.rsqrt(deg)
    dis_ref[...] = dis
    s0 = x_ref[...] * dis
    s2_ref[0] = s0[:, :FH]
    s2_ref[1] = s0[:, FH:]


_prep_call = pl.pallas_call(
    _prep_body,
    grid=(_G,),
    in_specs=[
        pl.BlockSpec((_R, 16), lambda i: (i, 0)),
        pl.BlockSpec((_R, 16), lambda i: (i + _O, 0)),
        pl.BlockSpec((_R, F), lambda i: (i, 0)),
    ],
    out_specs=[
        pl.BlockSpec((_R, 1), lambda i: (i, 0)),
        pl.BlockSpec((2, _R, FH), lambda i: (0, i, 0)),
    ],
    out_shape=[
        jax.ShapeDtypeStruct((N, 1), jnp.float32),
        jax.ShapeDtypeStruct((2, NPA, FH), jnp.float32),
    ],
)


def _layer_body(plo_ref, phi_ref, dis_ref, w_ref, y_ref, snext_ref):
    agg = jnp.concatenate([plo_ref[...], phi_ref[...]], axis=1)
    a = agg * dis_ref[...]
    y = jnp.dot(a, w_ref[...], preferred_element_type=jnp.float32)
    y_ref[...] = y
    s = jnp.maximum(y, 0.0) * dis_ref[...]
    snext_ref[0] = s[:, :FH]
    snext_ref[1] = s[:, FH:]


_layer_call = pl.pallas_call(
    _layer_body,
    grid=(_G,),
    in_specs=[
        pl.BlockSpec((_R, FH), lambda i: (i, 0)),
        pl.BlockSpec((_R, FH), lambda i: (i + _O, 0)),
        pl.BlockSpec((_R, 1), lambda i: (i, 0)),
        pl.BlockSpec((F, F), lambda i: (0, 0)),
    ],
    out_specs=[
        pl.BlockSpec((_R, F), lambda i: (i, 0)),
        pl.BlockSpec((2, _R, FH), lambda i: (0, i, 0)),
    ],
    out_shape=[
        jax.ShapeDtypeStruct((N, F), jnp.float32),
        jax.ShapeDtypeStruct((2, NPA, FH), jnp.float32),
    ],
)


def kernel(graph, features, W1, W2, W3):
    srcp = jnp.pad(graph[0], (0, EP - E))
    padv = N + jnp.arange(EP - E, dtype=jnp.int32) % (NPA - N)
    dstp = jnp.concatenate([graph[1], padv])
    src2 = srcp.reshape(NCHUNK, CH)
    dst2 = dstp.reshape(NCHUNK, CH)
    ones_d = jnp.ones((CH, 16), jnp.float32)
    zeros_d = jnp.zeros((RPT, 16), jnp.float32)
    zeros_f = jnp.zeros((RPT, FH), jnp.float32)
    deg = _deg_call(dst2, ones_d, zeros_d)
    dis, s2 = _prep_call(deg, deg, features)

    def step(carry, W):
        s2, _ = carry
        part = _agg_call(src2, dst2, s2.reshape(2 * NPA, FH), zeros_f)
        y, s2_next = _layer_call(part, part, dis, W)
        return (s2_next, y), None

    y0 = jnp.zeros((N, F), jnp.float32)
    (_, y), _ = lax.scan(step, (s2, y0), jnp.stack([W1, W2, W3]))
    return y


# speedup vs baseline: 1.8675x; 1.0098x over previous
import jax
import jax.numpy as jnp
from jax import lax
from jax.experimental import pallas as pl
from jax.experimental.pallas import tpu as pltpu
from jax.experimental.pallas import tpu_sc as plsc

N = 10000
E = 320000
F = 128
FH = F // 2

NC = 2
NS = 16
NW = NC * NS
CH = 128
NBUF = 3
CPT = 162
NCHUNK = NS * CPT
EP = NCHUNK * CH
DCPT = NCHUNK // NW
NP = 12800
NPA = 10112
RPT = NPA // NS

_mesh = plsc.VectorSubcoreMesh(core_axis_name="c", subcore_axis_name="s",
                               num_cores=NC, num_subcores=NS)


def _deg_body(dst_hbm, ones_hbm, zeros_hbm, deg_hbm,
              idx_v, ones_v, acc_sh):
    c = lax.axis_index("c")
    s = lax.axis_index("s")
    w = c * NS + s
    pltpu.sync_copy(zeros_hbm, acc_sh.at[pl.ds(s * RPT, RPT)])
    pltpu.sync_copy(ones_hbm, ones_v)
    pltpu.sync_copy(dst_hbm.at[pl.ds(w * DCPT, DCPT)], idx_v)
    plsc.subcore_barrier()

    def step(t, carry):
        pltpu.sync_copy(ones_v, acc_sh.at[idx_v.at[t]], add=True)
        return carry

    lax.fori_loop(0, DCPT, step, 0)
    plsc.subcore_barrier()
    pltpu.sync_copy(acc_sh.at[pl.ds(s * RPT, RPT)],
                    deg_hbm.at[pl.ds(c * NP + s * RPT, RPT)])


def _agg_body(src_hbm, dst_hbm, x_hbm, zeros_hbm, part_hbm,
              src_v, d0, d1, d2, r0, r1, r2, x_sh, acc_sh,
              gs0, gs1, gs2, ss0, ss1, ss2, ds0, ds1, ds2):
    c = lax.axis_index("c")
    s = lax.axis_index("s")
    rows = [r0, r1, r2]
    dsts = [d0, d1, d2]
    gsems = [gs0, gs1, gs2]
    ssems = [ss0, ss1, ss2]
    dsems = [ds0, ds1, ds2]
    pltpu.sync_copy(zeros_hbm, acc_sh.at[pl.ds(s * RPT, RPT)])
    pltpu.sync_copy(x_hbm.at[pl.ds(c * NPA + s * RPT, RPT)],
                    x_sh.at[pl.ds(s * RPT, RPT)])
    pltpu.sync_copy(src_hbm.at[pl.ds(s * CPT, CPT)], src_v)
    plsc.subcore_barrier()
    tbase = s * CPT

    def gather(t, b):
        return pltpu.make_async_copy(x_sh.at[src_v.at[t]], rows[b], gsems[b])

    def dload(t, b):
        return pltpu.make_async_copy(dst_hbm.at[tbase + t], dsts[b], dsems[b])

    def scat_start(t, b):
        pltpu.async_copy(rows[b], acc_sh.at[dsts[b]], ssems[b], add=True)

    def scat_wait(t, b):
        pltpu.make_async_copy(rows[b], acc_sh.at[dsts[b]], ssems[b]).wait()

    for t in range(2):
        dload(t, t).start()
        gather(t, t).start()
    for t in range(NBUF):
        gather(t, t).wait()
        dload(t, t).wait()
        scat_start(t, t)
        if t >= 1:
            scat_wait(t - 1, t - 1)
        dload(t + 2, (t + 2) % NBUF).start()
        gather(t + 2, (t + 2) % NBUF).start()

    def step(g, carry):
        for b in range(NBUF):
            t = g * NBUF + b
            gather(t, b).wait()
            dload(t, b).wait()
            scat_start(t, b)
            scat_wait(t - 1, (b - 1) % NBUF)
            dload(t + 2, (b + 2) % NBUF).start()
            gather(t + 2, (b + 2) % NBUF).start()
        return carry

    lax.fori_loop(1, CPT // NBUF - 1, step, 0)
    for b in range(NBUF):
        t = CPT - NBUF + b
        gather(t, b).wait()
        dload(t, b).wait()
        scat_start(t, b)
        scat_wait(t - 1, (b - 1) % NBUF)
        if t + 2 <= CPT - 1:
            dload(t + 2, (b + 2) % NBUF).start()
            gather(t + 2, (b + 2) % NBUF).start()
    scat_wait(CPT - 1, (CPT - 1) % NBUF)

    plsc.subcore_barrier()
    pltpu.sync_copy(acc_sh.at[pl.ds(s * RPT, RPT)],
                    part_hbm.at[pl.ds(c * NP + s * RPT, RPT)])


_sc_params = pltpu.CompilerParams(use_tc_tiling_on_sc=False)

_deg_call = pl.kernel(
    _deg_body,
    out_type=jax.ShapeDtypeStruct((2 * NP, 16), jnp.float32),
    mesh=_mesh,
    compiler_params=_sc_params,
    scratch_types=[
        pltpu.VMEM((DCPT, CH), jnp.int32),
        pltpu.VMEM((CH, 16), jnp.float32),
        pltpu.VMEM_SHARED((NPA, 16), jnp.float32),
    ],
)

_agg_call = pl.kernel(
    _agg_body,
    out_type=jax.ShapeDtypeStruct((2 * NP, FH), jnp.float32),
    mesh=_mesh,
    compiler_params=_sc_params,
    scratch_types=[
        pltpu.VMEM((CPT, CH), jnp.int32),
        pltpu.VMEM((CH,), jnp.int32),
        pltpu.VMEM((CH,), jnp.int32),
        pltpu.VMEM((CH,), jnp.int32),
        pltpu.VMEM((CH, FH), jnp.float32),
        pltpu.VMEM((CH, FH), jnp.float32),
        pltpu.VMEM((CH, FH), jnp.float32),
        pltpu.VMEM_SHARED((NPA, FH), jnp.float32),
        pltpu.VMEM_SHARED((NPA, FH), jnp.float32),
        pltpu.SemaphoreType.DMA,
        pltpu.SemaphoreType.DMA,
        pltpu.SemaphoreType.DMA,
        pltpu.SemaphoreType.DMA,
        pltpu.SemaphoreType.DMA,
        pltpu.SemaphoreType.DMA,
        pltpu.SemaphoreType.DMA,
        pltpu.SemaphoreType.DMA,
        pltpu.SemaphoreType.DMA,
    ],
)


_R = 400
_G = N // _R
_O = NP // _R


def _prep_body(d0_ref, d1_ref, x_ref, dis_ref, s2_ref):
    dsum = d0_ref[...] + d1_ref[...]
    deg = jnp.maximum(dsum[:, 0:1], 1.0)
    dis = lax.rsqrt(deg)
    dis_ref[...] = dis
    s0 = x_ref[...] * dis
    s2_ref[0] = s0[:, :FH]
    s2_ref[1] = s0[:, FH:]


_prep_call = pl.pallas_call(
    _prep_body,
    grid=(_G,),
    in_specs=[
        pl.BlockSpec((_R, 16), lambda i: (i, 0)),
        pl.BlockSpec((_R, 16), lambda i: (i + _O, 0)),
        pl.BlockSpec((_R, F), lambda i: (i, 0)),
    ],
    out_specs=[
        pl.BlockSpec((_R, 1), lambda i: (i, 0)),
        pl.BlockSpec((2, _R, FH), lambda i: (0, i, 0)),
    ],
    out_shape=[
        jax.ShapeDtypeStruct((N, 1), jnp.float32),
        jax.ShapeDtypeStruct((2, NPA, FH), jnp.float32),
    ],
)


def _layer_body(plo_ref, phi_ref, dis_ref, w_ref, y_ref, snext_ref):
    agg = jnp.concatenate([plo_ref[...], phi_ref[...]], axis=1)
    a = agg * dis_ref[...]
    y = jnp.dot(a, w_ref[...], preferred_element_type=jnp.float32)
    y_ref[...] = y
    s = jnp.maximum(y, 0.0) * dis_ref[...]
    snext_ref[0] = s[:, :FH]
    snext_ref[1] = s[:, FH:]


_layer_call = pl.pallas_call(
    _layer_body,
    grid=(_G,),
    in_specs=[
        pl.BlockSpec((_R, FH), lambda i: (i, 0)),
        pl.BlockSpec((_R, FH), lambda i: (i + _O, 0)),
        pl.BlockSpec((_R, 1), lambda i: (i, 0)),
        pl.BlockSpec((F, F), lambda i: (0, 0)),
    ],
    out_specs=[
        pl.BlockSpec((_R, F), lambda i: (i, 0)),
        pl.BlockSpec((2, _R, FH), lambda i: (0, i, 0)),
    ],
    out_shape=[
        jax.ShapeDtypeStruct((N, F), jnp.float32),
        jax.ShapeDtypeStruct((2, NPA, FH), jnp.float32),
    ],
)


def kernel(graph, features, W1, W2, W3):
    srcp = jnp.pad(graph[0], (0, EP - E))
    padv = N + jnp.arange(EP - E, dtype=jnp.int32) % (NPA - N)
    dstp = jnp.concatenate([graph[1], padv])
    src2 = srcp.reshape(NCHUNK, CH)
    dst2 = dstp.reshape(NCHUNK, CH)
    ones_d = jnp.ones((CH, 16), jnp.float32)
    zeros_d = jnp.zeros((RPT, 16), jnp.float32)
    zeros_f = jnp.zeros((RPT, FH), jnp.float32)
    deg = _deg_call(dst2, ones_d, zeros_d)
    dis, s2 = _prep_call(deg, deg, features)

    def step(carry, W):
        s2, _ = carry
        part = _agg_call(src2, dst2, s2.reshape(2 * NPA, FH), zeros_f)
        y, s2_next = _layer_call(part, part, dis, W)
        return (s2_next, y), None

    y0 = jnp.zeros((N, F), jnp.float32)
    (_, y), _ = lax.scan(step, (s2, y0), jnp.stack([W1, W2, W3]), unroll=3)
    return y
